# baseline (speedup 1.0000x reference)
import functools

import jax
import jax.numpy as jnp
from jax import lax
from jax.experimental import pallas as pl
from jax.experimental.pallas import tpu as pltpu
from jax.experimental.pallas import tpu_sc as plsc

B, D, H, E, C = 4096, 1024, 1024, 8, 1024
NC, NS, L = 2, 16, 16
NW = NC * NS
CHUNK = B // NW
RG = CHUNK // L
TM = 256
TMS = 8
NTP = B // TM + E - 1
BP = NTP * TM
assert NTP <= 2 * L


def _mesh():
    return plsc.VectorSubcoreMesh(
        core_axis_name="c", subcore_axis_name="s", num_cores=NC, num_subcores=NS
    )


def _wid():
    return lax.axis_index("s") * NC + lax.axis_index("c")


def _logits_body(x_ref, wr_ref, br_ref, out_ref):
    out_ref[...] = (
        jnp.dot(x_ref[...], wr_ref[...], preferred_element_type=jnp.float32)
        + br_ref[...]
    )


def _feat_body(x_ref, wf_ref, bf_ref, out_ref):
    f = jnp.dot(x_ref[...], wf_ref[...], preferred_element_type=jnp.float32)
    out_ref[...] = jnp.maximum(f + bf_ref[...], 0.0)


def _ra_body(logits_hbm, choices_hbm, counts_hbm, log_v, ch_v, cnt_v):
    base = _wid() * CHUNK
    pltpu.sync_copy(logits_hbm.at[pl.ds(base, CHUNK)], log_v)
    lane = lax.iota(jnp.int32, L)
    ch_regs = []
    for j in range(RG):
        toks = jnp.full((L,), j * L, jnp.int32) + lane
        best = plsc.load_gather(log_v, [toks, jnp.zeros((L,), jnp.int32)])
        arg = jnp.zeros((L,), jnp.int32)
        for e in range(1, E):
            v = plsc.load_gather(log_v, [toks, jnp.full((L,), e, jnp.int32)])
            upd = v > best
            arg = jnp.where(upd, e, arg)
            best = jnp.where(upd, v, best)
        ch_regs.append(arg)
        ch_v[pl.ds(j * L, L)] = arg
    cnts = jnp.zeros((L,), jnp.int32)
    for j in range(RG):
        for e in range(E):
            c = plsc.all_reduce_population_count(ch_regs[j] == e)
            cnts = cnts + jnp.where(lane == e, c, 0)
    cnt_v[...] = cnts
    pltpu.sync_copy(ch_v, choices_hbm.at[pl.ds(base, CHUNK)])
    pltpu.sync_copy(cnt_v, counts_hbm.at[_wid()])


_SCAT = 32
_NB = CHUNK // _SCAT


def _rb_body(choices_hbm, counts_hbm, feat_hbm,
             inv_hbm, sf_hbm, wk_hbm,
             ch_v, cnts_v, inv_v, wk_v, idx0, idx1, idx2, idx3,
             fbuf, sem):
    wid = _wid()
    base = wid * CHUNK
    pltpu.sync_copy(choices_hbm.at[pl.ds(base, CHUNK)], ch_v)
    pltpu.sync_copy(counts_hbm, cnts_v)
    lane = lax.iota(jnp.int32, L)
    totals = jnp.zeros((L,), jnp.int32)
    prefix = jnp.zeros((L,), jnp.int32)
    for w in range(NW):
        row = cnts_v[w, :]
        totals = totals + row
        prefix = prefix + jnp.where(w < wid, row, 0)
    pcnt = lax.shift_left(
        lax.shift_right_logical(totals + (TM - 1), TMS), TMS)
    ebase = plsc.cumsum(pcnt) - pcnt

    running = ebase + prefix
    idx_bufs = (idx0, idx1, idx2, idx3)
    for j in range(RG):
        v = ch_v[pl.ds(j * L, L)]
        dest = jnp.zeros((L,), jnp.int32)
        for e in range(E):
            m = v == e
            ones = m.astype(jnp.int32)
            rank = plsc.cumsum(ones) - 1
            base_e = jnp.sum(jnp.where(lane == e, running, 0))
            dest = jnp.where(m, base_e + rank, dest)
            c = plsc.all_reduce_population_count(m)
            running = running + jnp.where(lane == e, c, 0)
        inv_v[pl.ds(j * L, L)] = dest
        idx_bufs[j // 2][pl.ds((j % 2) * L, L)] = dest
    pltpu.sync_copy(inv_v, inv_hbm.at[pl.ds(base, CHUNK)])
    for k in range(_NB):
        pltpu.sync_copy(feat_hbm.at[pl.ds(base + k * _SCAT, _SCAT)], fbuf)
        pltpu.async_copy(fbuf, sf_hbm.at[idx_bufs[k]], sem).wait()

    @pl.when(wid == 0)
    def _():
        for half in range(2):
            tstart = (lane + half * L) * TM
            emap = jnp.full((L,), -1, jnp.int32)
            for e in range(E + 1):
                off_e = jnp.sum(jnp.where(lane == e, ebase, 0))
                emap = emap + (off_e <= tstart).astype(jnp.int32)
            wk_v[pl.ds(half * L, L)] = jnp.minimum(emap, E - 1)
        pltpu.sync_copy(wk_v, wk_hbm)


def _g_body(wk_r, fs_ref, lw_ref, lb_ref, out_ref):
    out_ref[...] = (
        jnp.dot(fs_ref[...].astype(jnp.bfloat16), lw_ref[0],
                preferred_element_type=jnp.float32)
        + lb_ref[0]
    )


def _u_body(so_hbm, inv_hbm, out_hbm, idx_v, buf, sem):
    base = _wid() * CHUNK
    for k in range(_NB):
        pltpu.sync_copy(inv_hbm.at[pl.ds(base + k * _SCAT, _SCAT)], idx_v)
        pltpu.async_copy(so_hbm.at[idx_v], buf, sem).wait()
        pltpu.sync_copy(buf, out_hbm.at[pl.ds(base + k * _SCAT, _SCAT)])


def kernel(inputs, W_feat, b_feat, W_route, b_route, leaf_W, leaf_b):
    logits = pl.pallas_call(
        _logits_body,
        grid=(4,),
        in_specs=[
            pl.BlockSpec((B // 4, D), lambda i: (i, 0)),
            pl.BlockSpec((D, E), lambda i: (0, 0)),
            pl.BlockSpec((1, E), lambda i: (0, 0)),
        ],
        out_specs=pl.BlockSpec((B // 4, E), lambda i: (i, 0)),
        out_shape=jax.ShapeDtypeStruct((B, E), jnp.float32),
    )(inputs, W_route, b_route.reshape(1, E))

    feat = pl.pallas_call(
        _feat_body,
        grid=(8,),
        in_specs=[
            pl.BlockSpec((B // 8, D), lambda i: (i, 0)),
            pl.BlockSpec((D, H), lambda i: (0, 0)),
            pl.BlockSpec((1, H), lambda i: (0, 0)),
        ],
        out_specs=pl.BlockSpec((B // 8, H), lambda i: (i, 0)),
        out_shape=jax.ShapeDtypeStruct((B, H), jnp.float32),
    )(inputs, W_feat, b_feat.reshape(1, H))

    leaf_W_b = leaf_W.astype(jnp.bfloat16)

    choices, counts = pl.kernel(
        _ra_body,
        out_type=[
            jax.ShapeDtypeStruct((B,), jnp.int32),
            jax.ShapeDtypeStruct((NW, L), jnp.int32),
        ],
        mesh=_mesh(),
        compiler_params=pltpu.CompilerParams(needs_layout_passes=False),
        scratch_types=[
            pltpu.VMEM((CHUNK, E), jnp.float32),
            pltpu.VMEM((CHUNK,), jnp.int32),
            pltpu.VMEM((L,), jnp.int32),
        ],
    )(logits)

    inv, sorted_feat, wk = pl.kernel(
        _rb_body,
        out_type=[
            jax.ShapeDtypeStruct((B,), jnp.int32),
            jax.ShapeDtypeStruct((BP, H), jnp.float32),
            jax.ShapeDtypeStruct((2 * L,), jnp.int32),
        ],
        mesh=_mesh(),
        compiler_params=pltpu.CompilerParams(needs_layout_passes=False),
        scratch_types=[
            pltpu.VMEM((CHUNK,), jnp.int32),
            pltpu.VMEM((NW, L), jnp.int32),
            pltpu.VMEM((CHUNK,), jnp.int32),
            pltpu.VMEM((2 * L,), jnp.int32),
            pltpu.VMEM((_SCAT,), jnp.int32),
            pltpu.VMEM((_SCAT,), jnp.int32),
            pltpu.VMEM((_SCAT,), jnp.int32),
            pltpu.VMEM((_SCAT,), jnp.int32),
            pltpu.VMEM((_SCAT, H), jnp.float32),
            pltpu.SemaphoreType.DMA,
        ],
    )(choices, counts, feat)

    sorted_out = pl.pallas_call(
        _g_body,
        grid_spec=pltpu.PrefetchScalarGridSpec(
            num_scalar_prefetch=1,
            grid=(NTP,),
            in_specs=[
                pl.BlockSpec((TM, H), lambda w, wk: (w, 0)),
                pl.BlockSpec((1, H, C), lambda w, wk: (wk[w], 0, 0)),
                pl.BlockSpec((1, 1, C), lambda w, wk: (wk[w], 0, 0)),
            ],
            out_specs=pl.BlockSpec((TM, C), lambda w, wk: (w, 0)),
        ),
        out_shape=jax.ShapeDtypeStruct((BP, C), jnp.float32),
    )(wk, sorted_feat, leaf_W_b, leaf_b.reshape(E, 1, C))

    predictions = pl.kernel(
        _u_body,
        out_type=jax.ShapeDtypeStruct((B, C), jnp.float32),
        mesh=_mesh(),
        compiler_params=pltpu.CompilerParams(needs_layout_passes=False),
        scratch_types=[
            pltpu.VMEM((_SCAT,), jnp.int32),
            pltpu.VMEM((_SCAT, C), jnp.float32),
            pltpu.SemaphoreType.DMA,
        ],
    )(sorted_out, inv)
    return predictions


# speedup vs baseline: 1.1563x; 1.1563x over previous
import functools

import jax
import jax.numpy as jnp
from jax import lax
from jax.experimental import pallas as pl
from jax.experimental.pallas import tpu as pltpu
from jax.experimental.pallas import tpu_sc as plsc

B, D, H, E, C = 4096, 1024, 1024, 8, 1024
NC, NS, L = 2, 16, 16
NW = NC * NS
CHUNK = B // NW
RG = CHUNK // L
TM = 256
NT = B // TM
W = NT + E - 1
WK = 6 * L
assert NT == L


def _mesh():
    return plsc.VectorSubcoreMesh(
        core_axis_name="c", subcore_axis_name="s", num_cores=NC, num_subcores=NS
    )


def _wid():
    return lax.axis_index("s") * NC + lax.axis_index("c")


def _logits_body(x_ref, wr_ref, br_ref, out_ref):
    out_ref[...] = (
        jnp.dot(x_ref[...], wr_ref[...], preferred_element_type=jnp.float32)
        + br_ref[...]
    )


def _ra_body(logits_hbm, choices_hbm, counts_hbm, log_v, ch_v, cnt_v):
    base = _wid() * CHUNK
    pltpu.sync_copy(logits_hbm.at[pl.ds(base, CHUNK)], log_v)
    lane = lax.iota(jnp.int32, L)
    ch_regs = []
    for j in range(RG):
        toks = jnp.full((L,), j * L, jnp.int32) + lane
        best = plsc.load_gather(log_v, [toks, jnp.zeros((L,), jnp.int32)])
        arg = jnp.zeros((L,), jnp.int32)
        for e in range(1, E):
            v = plsc.load_gather(log_v, [toks, jnp.full((L,), e, jnp.int32)])
            upd = v > best
            arg = jnp.where(upd, e, arg)
            best = jnp.where(upd, v, best)
        ch_regs.append(arg)
        ch_v[pl.ds(j * L, L)] = arg
    cnts = jnp.zeros((L,), jnp.int32)
    for j in range(RG):
        for e in range(E):
            c = plsc.all_reduce_population_count(ch_regs[j] == e)
            cnts = cnts + jnp.where(lane == e, c, 0)
    cnt_v[...] = cnts
    pltpu.sync_copy(ch_v, choices_hbm.at[pl.ds(base, CHUNK)])
    pltpu.sync_copy(cnt_v, counts_hbm.at[_wid()])


_SCAT = 32
_NB = CHUNK // _SCAT


def _rb_body(choices_hbm, counts_hbm, x_hbm,
             inv_hbm, sx_hbm, wk_hbm,
             ch_v, cnts_v, inv_v, wk_v, idx0, idx1, idx2, idx3,
             xbuf0, xbuf1, sem0, sem1):
    wid = _wid()
    base = wid * CHUNK
    pltpu.sync_copy(choices_hbm.at[pl.ds(base, CHUNK)], ch_v)
    pltpu.sync_copy(counts_hbm, cnts_v)
    lane = lax.iota(jnp.int32, L)
    totals = jnp.zeros((L,), jnp.int32)
    prefix = jnp.zeros((L,), jnp.int32)
    for w in range(NW):
        row = cnts_v[w, :]
        totals = totals + row
        prefix = prefix + jnp.where(w < wid, row, 0)
    ebase = plsc.cumsum(totals) - totals

    running = ebase + prefix
    idx_bufs = (idx0, idx1, idx2, idx3)
    for j in range(RG):
        v = ch_v[pl.ds(j * L, L)]
        dest = jnp.zeros((L,), jnp.int32)
        for e in range(E):
            m = v == e
            ones = m.astype(jnp.int32)
            rank = plsc.cumsum(ones) - 1
            base_e = jnp.sum(jnp.where(lane == e, running, 0))
            dest = jnp.where(m, base_e + rank, dest)
            c = plsc.all_reduce_population_count(m)
            running = running + jnp.where(lane == e, c, 0)
        inv_v[pl.ds(j * L, L)] = dest
        idx_bufs[j // 2][pl.ds((j % 2) * L, L)] = dest
    pltpu.sync_copy(inv_v, inv_hbm.at[pl.ds(base, CHUNK)])
    xbufs = (xbuf0, xbuf1)
    sems = (sem0, sem1)
    copies = []
    for k in range(_NB):
        if len(copies) == 2:
            copies.pop(0).wait()
        pltpu.sync_copy(x_hbm.at[pl.ds(base + k * _SCAT, _SCAT)], xbufs[k % 2])
        copies.append(
            pltpu.async_copy(xbufs[k % 2], sx_hbm.at[idx_bufs[k]], sems[k % 2]))
    for cp in copies:
        cp.wait()

    @pl.when(wid == 0)
    def _():
        tstart = lane * TM
        e_lo = jnp.full((L,), -1, jnp.int32)
        e_hi = jnp.full((L,), -1, jnp.int32)
        for e in range(E + 1):
            off_e = jnp.sum(jnp.where(lane == e, ebase, 0))
            e_lo = e_lo + (off_e <= tstart).astype(jnp.int32)
            e_hi = e_hi + (off_e <= tstart + (TM - 1)).astype(jnp.int32)
        cntv = e_hi - e_lo + 1
        startsv = plsc.cumsum(cntv) - cntv
        n_items = jnp.sum(cntv)
        for half in range(2):
            wv = lane + half * L
            tmap = jnp.full((L,), -1, jnp.int32)
            for t in range(NT):
                s_t = jnp.sum(jnp.where(lane == t, startsv, 0))
                tmap = tmap + (s_t <= wv).astype(jnp.int32)
            emap = jnp.zeros((L,), jnp.int32)
            for t in range(NT):
                s_t = jnp.sum(jnp.where(lane == t, startsv, 0))
                lo_t = jnp.sum(jnp.where(lane == t, e_lo, 0))
                hi_t = jnp.sum(jnp.where(lane == t, e_hi, 0))
                emap = jnp.where(tmap == t,
                                 jnp.minimum(lo_t + (wv - s_t), hi_t), emap)
            wk_v[pl.ds(half * L, L)] = tmap
            wk_v[pl.ds(2 * L + half * L, L)] = emap
        wk_v[pl.ds(4 * L, L)] = ebase
        wk_v[pl.ds(5 * L, L)] = jnp.zeros((L,), jnp.int32) + n_items
        pltpu.sync_copy(wk_v, wk_hbm)


def _g_body(wk_r, xs_ref, wf_ref, bf_ref, lw_ref, lb_ref, out_ref, feat_scr):
    w = pl.program_id(0)
    t = wk_r[w]
    e = wk_r[2 * L + w]
    prev_t = wk_r[jnp.maximum(w - 1, 0)]
    first = jnp.logical_or(w == 0, t != prev_t)

    @pl.when(first)
    def _():
        f = jnp.dot(xs_ref[...], wf_ref[...], preferred_element_type=jnp.float32)
        feat_scr[...] = jnp.maximum(f + bf_ref[...], 0.0)

    p = t * TM + lax.broadcasted_iota(jnp.int32, (TM, 1), 0)
    mask = (p >= wk_r[4 * L + e]) & (p < wk_r[4 * L + e + 1]) & (w < wk_r[5 * L])
    contrib = jnp.dot(feat_scr[...], lw_ref[0], preferred_element_type=jnp.float32)
    contrib = jnp.where(mask, contrib + lb_ref[0], 0.0)

    @pl.when(first)
    def _():
        out_ref[...] = contrib

    @pl.when(jnp.logical_not(first))
    def _():
        out_ref[...] += contrib


def _u_body(so_hbm, inv_hbm, out_hbm, idx_v, buf0, buf1, sem0, sem1):
    base = _wid() * CHUNK
    bufs = (buf0, buf1)
    sems = (sem0, sem1)
    pltpu.sync_copy(inv_hbm.at[pl.ds(base, CHUNK)], idx_v)
    copies = []
    for k in range(_NB):
        if len(copies) == 2:
            k0, cp = copies.pop(0)
            cp.wait()
            pltpu.sync_copy(bufs[k0 % 2],
                            out_hbm.at[pl.ds(base + k0 * _SCAT, _SCAT)])
        copies.append(
            (k, pltpu.async_copy(so_hbm.at[idx_v.at[pl.ds(k * _SCAT, _SCAT)]],
                                 bufs[k % 2], sems[k % 2])))
    for k0, cp in copies:
        cp.wait()
        pltpu.sync_copy(bufs[k0 % 2],
                        out_hbm.at[pl.ds(base + k0 * _SCAT, _SCAT)])


def kernel(inputs, W_feat, b_feat, W_route, b_route, leaf_W, leaf_b):
    logits = pl.pallas_call(
        _logits_body,
        grid=(4,),
        in_specs=[
            pl.BlockSpec((B // 4, D), lambda i: (i, 0)),
            pl.BlockSpec((D, E), lambda i: (0, 0)),
            pl.BlockSpec((1, E), lambda i: (0, 0)),
        ],
        out_specs=pl.BlockSpec((B // 4, E), lambda i: (i, 0)),
        out_shape=jax.ShapeDtypeStruct((B, E), jnp.float32),
    )(inputs, W_route, b_route.reshape(1, E))

    choices, counts = pl.kernel(
        _ra_body,
        out_type=[
            jax.ShapeDtypeStruct((B,), jnp.int32),
            jax.ShapeDtypeStruct((NW, L), jnp.int32),
        ],
        mesh=_mesh(),
        compiler_params=pltpu.CompilerParams(needs_layout_passes=False),
        scratch_types=[
            pltpu.VMEM((CHUNK, E), jnp.float32),
            pltpu.VMEM((CHUNK,), jnp.int32),
            pltpu.VMEM((L,), jnp.int32),
        ],
    )(logits)

    inv, sorted_x, wk = pl.kernel(
        _rb_body,
        out_type=[
            jax.ShapeDtypeStruct((B,), jnp.int32),
            jax.ShapeDtypeStruct((B, D), jnp.float32),
            jax.ShapeDtypeStruct((WK,), jnp.int32),
        ],
        mesh=_mesh(),
        compiler_params=pltpu.CompilerParams(needs_layout_passes=False),
        scratch_types=[
            pltpu.VMEM((CHUNK,), jnp.int32),
            pltpu.VMEM((NW, L), jnp.int32),
            pltpu.VMEM((CHUNK,), jnp.int32),
            pltpu.VMEM((WK,), jnp.int32),
            pltpu.VMEM((_SCAT,), jnp.int32),
            pltpu.VMEM((_SCAT,), jnp.int32),
            pltpu.VMEM((_SCAT,), jnp.int32),
            pltpu.VMEM((_SCAT,), jnp.int32),
            pltpu.VMEM((_SCAT, D), jnp.float32),
            pltpu.VMEM((_SCAT, D), jnp.float32),
            pltpu.SemaphoreType.DMA,
            pltpu.SemaphoreType.DMA,
        ],
    )(choices, counts, inputs)

    sorted_out = pl.pallas_call(
        _g_body,
        grid_spec=pltpu.PrefetchScalarGridSpec(
            num_scalar_prefetch=1,
            grid=(W,),
            in_specs=[
                pl.BlockSpec((TM, D), lambda w, wk: (wk[w], 0)),
                pl.BlockSpec((D, H), lambda w, wk: (0, 0)),
                pl.BlockSpec((1, H), lambda w, wk: (0, 0)),
                pl.BlockSpec((1, H, C), lambda w, wk: (wk[2 * L + w], 0, 0)),
                pl.BlockSpec((1, 1, C), lambda w, wk: (wk[2 * L + w], 0, 0)),
            ],
            out_specs=pl.BlockSpec((TM, C), lambda w, wk: (wk[w], 0)),
            scratch_shapes=[pltpu.VMEM((TM, H), jnp.float32)],
        ),
        out_shape=jax.ShapeDtypeStruct((B, C), jnp.float32),
    )(wk, sorted_x, W_feat, b_feat.reshape(1, H),
      leaf_W, leaf_b.reshape(E, 1, C))

    predictions = pl.kernel(
        _u_body,
        out_type=jax.ShapeDtypeStruct((B, C), jnp.float32),
        mesh=_mesh(),
        compiler_params=pltpu.CompilerParams(needs_layout_passes=False),
        scratch_types=[
            pltpu.VMEM((CHUNK,), jnp.int32),
            pltpu.VMEM((_SCAT, C), jnp.float32),
            pltpu.VMEM((_SCAT, C), jnp.float32),
            pltpu.SemaphoreType.DMA,
            pltpu.SemaphoreType.DMA,
        ],
    )(sorted_out, inv)
    return predictions
